# baseline (speedup 1.0000x reference)
import functools

import jax
import jax.numpy as jnp
from jax import lax
from jax.experimental import pallas as pl
from jax.experimental.pallas import tpu as pltpu
from jax.experimental.pallas import tpu_sc as plsc

_EMB = 64
_NC = 2
_NS = 16
_NW = _NC * _NS

_CHUNK = 256
_SUB = 128
_NSUB = _CHUNK // _SUB
_NBUF = 5


def _sc_gather(t_flat, table, n_rows):
    b_per_w = n_rows // _NW
    n_chunks = b_per_w // _CHUNK
    n_quads = n_chunks // _NBUF

    mesh = plsc.VectorSubcoreMesh(core_axis_name="c", subcore_axis_name="s")

    @functools.partial(
        pl.kernel,
        mesh=mesh,
        out_type=jax.ShapeDtypeStruct((n_rows, _EMB), jnp.float32),
        scratch_types=[
            pltpu.VMEM((b_per_w,), jnp.int32),
        ] + [pltpu.VMEM((_CHUNK, _EMB), jnp.float32)] * _NBUF
          + [pltpu.SemaphoreType.DMA] * (2 * _NBUF),
        compiler_params=pltpu.CompilerParams(use_tc_tiling_on_sc=False),
    )
    def k(t_hbm, table_hbm, out_hbm, idx_v, *bufs_and_sems):
        rows = bufs_and_sems[:_NBUF]
        gs = bufs_and_sems[_NBUF:2 * _NBUF]
        os_ = bufs_and_sems[2 * _NBUF:3 * _NBUF]

        wid = lax.axis_index("s") * _NC + lax.axis_index("c")
        base = wid * b_per_w

        pltpu.sync_copy(t_hbm.at[pl.ds(base, b_per_w)], idx_v)

        def fire_gather(c, b):
            for j in range(_NSUB):
                pltpu.async_copy(
                    table_hbm.at[idx_v.at[pl.ds(c * _CHUNK + j * _SUB, _SUB)]],
                    rows[b].at[pl.ds(j * _SUB, _SUB)],
                    gs[b])

        def wait_gather(b):
            pltpu.make_async_copy(
                table_hbm.at[idx_v.at[pl.ds(0, _SUB)]],
                rows[b], gs[b]).wait()

        def fire_wb(c, b):
            pltpu.async_copy(
                rows[b], out_hbm.at[pl.ds(base + c * _CHUNK, _CHUNK)], os_[b])

        def wait_wb(b):
            pltpu.make_async_copy(
                rows[b], out_hbm.at[pl.ds(0, _CHUNK)], os_[b]).wait()

        for b in range(_NBUF):
            fire_gather(b, b)

        def body(i, carry):
            c0 = _NBUF * i
            for b in range(_NBUF):
                wait_gather(b)
                fire_wb(c0 + b, b)
            for b in range(_NBUF):
                wait_wb(b)
                fire_gather(c0 + _NBUF + b, b)
            return carry

        lax.fori_loop(0, n_quads - 1, body, 0)

        c_last = n_chunks - _NBUF
        for b in range(_NBUF):
            wait_gather(b)
            fire_wb(c_last + b, b)
        for b in range(_NBUF):
            wait_wb(b)

    return k(t_flat, table)


def kernel(t, pos_embeddings):
    b, s = t.shape
    flat = _sc_gather(t.reshape(-1), pos_embeddings, b * s)
    return flat.reshape(b, s, _EMB)
